# baseline (speedup 1.0000x reference)
import functools

import jax
import jax.numpy as jnp
from jax import lax
from jax.experimental import pallas as pl
from jax.experimental.pallas import tpu as pltpu
from jax.experimental.pallas import tpu_sc as plsc

_B = 256
_HE = 1024
_HD = 1024
_N = 32640
_T = 384

_WIN = 272
_NCK = _WIN // 16
_NW = 32
_SEG_PER_W = _B // _NW


def _bdot(a, b):
    return jnp.dot(a, b, preferred_element_type=jnp.float32)


def _p_kernel(prev_ref, w1t_ref, b_ref, hi_ref, lo_ref):
    p = _bdot(prev_ref[...].astype(jnp.bfloat16), w1t_ref[...]) + b_ref[...]
    hi = p.astype(jnp.bfloat16)
    hi_ref[...] = hi
    lo_ref[...] = (p - hi.astype(jnp.float32)).astype(jnp.bfloat16)


def _scores_kernel(starts_ref, ends_ref, enc_ref, w2t_ref, phi_ref, plo_ref,
                   v_ref, out_ref):
    t = pl.program_id(0)
    rows = t * _T + lax.broadcasted_iota(jnp.int32, (_T, 1), 0)
    in_seg = (rows >= starts_ref[...]) & (rows < ends_ref[...])
    oh = in_seg.astype(jnp.bfloat16)
    pre = _bdot(enc_ref[...].astype(jnp.bfloat16), w2t_ref[...])
    pre = pre + (_bdot(oh, phi_ref[...]) + _bdot(oh, plo_ref[...]))
    energy = jnp.tanh(pre).astype(jnp.bfloat16)
    out_ref[...] = _bdot(energy, v_ref[...])


def _sc_softmax_body(scores_hbm, offs_hbm, out_hbm, buf, val2, idx2, offs_v,
                     sem):
    w = lax.axis_index("s") * 2 + lax.axis_index("c")
    lane = lax.iota(jnp.int32, 16)

    wbase = pl.multiple_of(w * 16, 16)
    pltpu.sync_copy(offs_hbm.at[pl.ds(wbase, 16)], offs_v)
    ovec = offs_v[pl.ds(0, 16)]

    trash = jnp.full((16,), _N, jnp.int32)
    for c in range(3):
        for m in range(8):
            idx2[c, pl.ds(16 * m, 16)] = trash

    for k in range(_SEG_PER_W):
        start = ovec[k]
        end = ovec[8 + k]
        astart = pl.multiple_of(start & ~7, 8)
        pltpu.sync_copy(scores_hbm.at[pl.ds(astart, _WIN)],
                        buf.at[pl.ds(0, _WIN)])

        mvec = jnp.full((16,), -3.0e38, jnp.float32)
        for c in range(_NCK):
            gl = astart + 16 * c + lane
            valid = (gl >= start) & (gl < end)
            mvec = jnp.maximum(mvec,
                               jnp.where(valid, buf[pl.ds(16 * c, 16)],
                                         -3.0e38))
        m = mvec[0]
        for i in range(1, 16):
            m = jnp.maximum(m, mvec[i])

        svec = jnp.zeros((16,), jnp.float32)
        for c in range(_NCK):
            gl = astart + 16 * c + lane
            valid = (gl >= start) & (gl < end)
            ex = jnp.where(valid, jnp.exp(buf[pl.ds(16 * c, 16)] - m), 0.0)
            svec = svec + ex
            val2[c // 8, pl.ds(16 * (c % 8), 16)] = ex
            idx2[c // 8, pl.ds(16 * (c % 8), 16)] = jnp.where(valid, gl, _N)
        sm = svec[0]
        for i in range(1, 16):
            sm = sm + svec[i]
        invv = jnp.ones((16,), jnp.float32) / (jnp.zeros((16,), jnp.float32)
                                               + sm)

        for c in range(_NCK):
            val2[c // 8, pl.ds(16 * (c % 8), 16)] = (
                val2[c // 8, pl.ds(16 * (c % 8), 16)] * invv)

        for c in range(3):
            pltpu.async_copy(val2.at[c], out_hbm.at[idx2.at[c]], sem).wait()


_sc_softmax = functools.partial(
    pl.kernel,
    mesh=plsc.VectorSubcoreMesh(core_axis_name="c", subcore_axis_name="s"),
    out_type=jax.ShapeDtypeStruct((_N + 16,), jnp.float32),
    scratch_types=[
        pltpu.VMEM((_WIN,), jnp.float32),
        pltpu.VMEM((3, 128), jnp.float32),
        pltpu.VMEM((3, 128), jnp.int32),
        pltpu.VMEM((16,), jnp.int32),
        pltpu.SemaphoreType.DMA,
    ],
)(_sc_softmax_body)


def kernel(prev_hidden_states, encoder_output, tree_sizes, W, b, v):
    w1t = W[:, :_HD].T.astype(jnp.bfloat16)
    w2t = W[:, _HD:].T.astype(jnp.bfloat16)
    csum = jnp.cumsum(tree_sizes.astype(jnp.int32))
    offs = jnp.concatenate([jnp.zeros((1,), jnp.int32), csum])
    starts = offs[:_B].reshape(1, _B)
    ends = offs[1:_B + 1].reshape(1, _B)
    seg_of_w = (jnp.arange(_NW)[:, None] +
                _NW * jnp.arange(_SEG_PER_W)[None, :])
    woffs = jnp.concatenate(
        [offs[seg_of_w], offs[seg_of_w + 1]], axis=1).reshape(_NW * 16)
    b2 = b.reshape(1, _HE)
    v16 = v.reshape(_HE, 1).astype(jnp.bfloat16)

    p_hi, p_lo = pl.pallas_call(
        _p_kernel,
        out_shape=[
            jax.ShapeDtypeStruct((_B, _HE), jnp.bfloat16),
            jax.ShapeDtypeStruct((_B, _HE), jnp.bfloat16),
        ],
    )(prev_hidden_states, w1t, b2)

    grid = _N // _T
    scores = pl.pallas_call(
        _scores_kernel,
        grid=(grid,),
        in_specs=[
            pl.BlockSpec((1, _B), lambda t: (0, 0)),
            pl.BlockSpec((1, _B), lambda t: (0, 0)),
            pl.BlockSpec((_T, _HE), lambda t: (t, 0)),
            pl.BlockSpec((_HE, _HE), lambda t: (0, 0)),
            pl.BlockSpec((_B, _HE), lambda t: (0, 0)),
            pl.BlockSpec((_B, _HE), lambda t: (0, 0)),
            pl.BlockSpec((_HE, 1), lambda t: (0, 0)),
        ],
        out_specs=pl.BlockSpec((_T, 1), lambda t: (t, 0)),
        out_shape=jax.ShapeDtypeStruct((_N, 1), jnp.float32),
    )(starts, ends, encoder_output, w2t, p_hi, p_lo, v16)

    spad = jnp.concatenate(
        [scores.reshape(_N), jnp.zeros((32,), jnp.float32)])
    att = _sc_softmax(spad, woffs)
    return att[:_N].reshape(_N, 1)


# speedup vs baseline: 36.0623x; 36.0623x over previous
import functools

import jax
import jax.numpy as jnp
from jax import lax
from jax.experimental import pallas as pl
from jax.experimental.pallas import tpu as pltpu
from jax.experimental.pallas import tpu_sc as plsc

_B = 256
_HE = 1024
_HD = 1024
_N = 32640
_T = 384

_WIN = 272
_NCK = _WIN // 16
_NW = 32
_SEG_PER_W = _B // _NW


def _bdot(a, b):
    return jnp.dot(a, b, preferred_element_type=jnp.float32)


def _p_kernel(prev_ref, w1t_ref, b_ref, hi_ref, lo_ref):
    p = _bdot(prev_ref[...].astype(jnp.bfloat16), w1t_ref[...]) + b_ref[...]
    hi = p.astype(jnp.bfloat16)
    hi_ref[...] = hi
    lo_ref[...] = (p - hi.astype(jnp.float32)).astype(jnp.bfloat16)


def _scores_kernel(starts_ref, ends_ref, enc_ref, w2t_ref, phi_ref, plo_ref,
                   v_ref, out_ref):
    t = pl.program_id(0)
    rows = t * _T + lax.broadcasted_iota(jnp.int32, (_T, 1), 0)
    in_seg = (rows >= starts_ref[...]) & (rows < ends_ref[...])
    oh = in_seg.astype(jnp.bfloat16)
    pre = _bdot(enc_ref[...].astype(jnp.bfloat16), w2t_ref[...])
    pre = pre + (_bdot(oh, phi_ref[...]) + _bdot(oh, plo_ref[...]))
    energy = jnp.tanh(pre).astype(jnp.bfloat16)
    out_ref[...] = _bdot(energy, v_ref[...])


def _norm_kernel(starts_ref, ends_ref, s_ref, mx_ref, den_ref, out_ref):
    t = pl.program_id(0)
    rows = t * _T + lax.broadcasted_iota(jnp.int32, (_T, 1), 0)
    in_seg = (rows >= starts_ref[...]) & (rows < ends_ref[...])
    mrow = jnp.sum(jnp.where(in_seg, mx_ref[...], 0.0), axis=1, keepdims=True)
    drow = jnp.sum(jnp.where(in_seg, den_ref[...], 0.0), axis=1, keepdims=True)
    out_ref[...] = jnp.exp(s_ref[...] - mrow) / drow


def _sc_stats_body(scores_hbm, offs_hbm, out_hbm, buf, offs_v, stat_v):
    w = lax.axis_index("s") * 2 + lax.axis_index("c")
    lane = lax.iota(jnp.int32, 16)

    wbase = pl.multiple_of(w * 16, 16)
    pltpu.sync_copy(offs_hbm.at[pl.ds(wbase, 16)], offs_v)
    ovec = offs_v[pl.ds(0, 16)]

    mvals = jnp.zeros((16,), jnp.float32)
    svals = jnp.zeros((16,), jnp.float32)
    for k in range(_SEG_PER_W):
        start = ovec[k]
        end = ovec[8 + k]
        astart = pl.multiple_of(start & ~7, 8)
        pltpu.sync_copy(scores_hbm.at[pl.ds(astart, _WIN)],
                        buf.at[pl.ds(0, _WIN)])

        mvec = jnp.full((16,), -3.0e38, jnp.float32)
        for c in range(_NCK):
            gl = astart + 16 * c + lane
            valid = (gl >= start) & (gl < end)
            mvec = jnp.maximum(mvec,
                               jnp.where(valid, buf[pl.ds(16 * c, 16)],
                                         -3.0e38))
        m = mvec[0]
        for i in range(1, 16):
            m = jnp.maximum(m, mvec[i])

        svec = jnp.zeros((16,), jnp.float32)
        for c in range(_NCK):
            gl = astart + 16 * c + lane
            valid = (gl >= start) & (gl < end)
            svec = svec + jnp.where(
                valid, jnp.exp(buf[pl.ds(16 * c, 16)] - m), 0.0)
        sm = svec[0]
        for i in range(1, 16):
            sm = sm + svec[i]

        mvals = jnp.where(lane == k, jnp.zeros((16,), jnp.float32) + m, mvals)
        svals = jnp.where(lane == (8 + k),
                          jnp.zeros((16,), jnp.float32) + sm, svals)

    stat_v[pl.ds(0, 16)] = jnp.where(lane < 8, mvals,
                                     jnp.zeros((16,), jnp.float32))
    stat_v[pl.ds(0, 16)] = stat_v[pl.ds(0, 16)] + jnp.where(
        lane >= 8, svals, jnp.zeros((16,), jnp.float32))
    pltpu.sync_copy(stat_v, out_hbm.at[pl.ds(wbase, 16)])


_sc_stats = functools.partial(
    pl.kernel,
    mesh=plsc.VectorSubcoreMesh(core_axis_name="c", subcore_axis_name="s"),
    out_type=jax.ShapeDtypeStruct((_NW * 16,), jnp.float32),
    scratch_types=[
        pltpu.VMEM((_WIN,), jnp.float32),
        pltpu.VMEM((16,), jnp.int32),
        pltpu.VMEM((16,), jnp.float32),
    ],
)(_sc_stats_body)


def kernel(prev_hidden_states, encoder_output, tree_sizes, W, b, v):
    w1t = W[:, :_HD].T.astype(jnp.bfloat16)
    w2t = W[:, _HD:].T.astype(jnp.bfloat16)
    csum = jnp.cumsum(tree_sizes.astype(jnp.int32))
    offs = jnp.concatenate([jnp.zeros((1,), jnp.int32), csum])
    starts = offs[:_B].reshape(1, _B)
    ends = offs[1:_B + 1].reshape(1, _B)
    seg_of_w = (jnp.arange(_NW)[:, None] +
                _NW * jnp.arange(_SEG_PER_W)[None, :])
    woffs = jnp.concatenate(
        [offs[seg_of_w], offs[seg_of_w + 1]], axis=1).reshape(_NW * 16)
    b2 = b.reshape(1, _HE)
    v16 = v.reshape(_HE, 1).astype(jnp.bfloat16)

    p_hi, p_lo = pl.pallas_call(
        _p_kernel,
        out_shape=[
            jax.ShapeDtypeStruct((_B, _HE), jnp.bfloat16),
            jax.ShapeDtypeStruct((_B, _HE), jnp.bfloat16),
        ],
    )(prev_hidden_states, w1t, b2)

    grid = _N // _T
    scores = pl.pallas_call(
        _scores_kernel,
        grid=(grid,),
        in_specs=[
            pl.BlockSpec((1, _B), lambda t: (0, 0)),
            pl.BlockSpec((1, _B), lambda t: (0, 0)),
            pl.BlockSpec((_T, _HE), lambda t: (t, 0)),
            pl.BlockSpec((_HE, _HE), lambda t: (0, 0)),
            pl.BlockSpec((_B, _HE), lambda t: (0, 0)),
            pl.BlockSpec((_B, _HE), lambda t: (0, 0)),
            pl.BlockSpec((_HE, 1), lambda t: (0, 0)),
        ],
        out_specs=pl.BlockSpec((_T, 1), lambda t: (t, 0)),
        out_shape=jax.ShapeDtypeStruct((_N, 1), jnp.float32),
    )(starts, ends, encoder_output, w2t, p_hi, p_lo, v16)

    spad = jnp.concatenate(
        [scores.reshape(_N), jnp.zeros((32,), jnp.float32)])
    stats = _sc_stats(spad, woffs).reshape(_NW, 16)
    seg_perm = jnp.argsort(seg_of_w.reshape(-1))
    mx_row = stats[:, :8].reshape(-1)[seg_perm].reshape(1, _B)
    den_row = stats[:, 8:].reshape(-1)[seg_perm].reshape(1, _B)

    att = pl.pallas_call(
        _norm_kernel,
        grid=(grid,),
        in_specs=[
            pl.BlockSpec((1, _B), lambda t: (0, 0)),
            pl.BlockSpec((1, _B), lambda t: (0, 0)),
            pl.BlockSpec((_T, 1), lambda t: (t, 0)),
            pl.BlockSpec((1, _B), lambda t: (0, 0)),
            pl.BlockSpec((1, _B), lambda t: (0, 0)),
        ],
        out_specs=pl.BlockSpec((_T, 1), lambda t: (t, 0)),
        out_shape=jax.ShapeDtypeStruct((_N, 1), jnp.float32),
    )(starts, ends, scores, mx_row, den_row)
    return att


# speedup vs baseline: 40.4048x; 1.1204x over previous
import functools

import numpy as np

import jax
import jax.numpy as jnp
from jax import lax
from jax.experimental import pallas as pl
from jax.experimental.pallas import tpu as pltpu
from jax.experimental.pallas import tpu_sc as plsc

_B = 256
_HE = 1024
_HD = 1024
_N = 32640
_T = 640

_WIN = 272
_NCK = _WIN // 16
_NW = 32
_SEG_PER_W = _B // _NW
_SEG_PERM = np.argsort(
    (np.arange(_NW)[:, None] + _NW * np.arange(_SEG_PER_W)[None, :])
    .reshape(-1))


def _bdot(a, b):
    return jnp.dot(a, b, preferred_element_type=jnp.float32)


def _p_kernel(prev_ref, w1t_ref, b_ref, hi_ref, lo_ref):
    p = _bdot(prev_ref[...].astype(jnp.bfloat16), w1t_ref[...]) + b_ref[...]
    hi = p.astype(jnp.bfloat16)
    hi_ref[...] = hi
    lo_ref[...] = (p - hi.astype(jnp.float32)).astype(jnp.bfloat16)


def _scores_kernel(starts_ref, ends_ref, enc_ref, w2t_ref, phi_ref, plo_ref,
                   v_ref, out_ref):
    t = pl.program_id(0)
    rows = t * _T + lax.broadcasted_iota(jnp.int32, (_T, 1), 0)
    in_seg = (rows >= starts_ref[...]) & (rows < ends_ref[...])
    oh = in_seg.astype(jnp.bfloat16)
    pre = _bdot(enc_ref[...].astype(jnp.bfloat16), w2t_ref[...])
    pre = pre + (_bdot(oh, phi_ref[...]) + _bdot(oh, plo_ref[...]))
    energy = jnp.tanh(pre).astype(jnp.bfloat16)
    out_ref[...] = _bdot(energy, v_ref[...])


def _norm_kernel(starts_ref, ends_ref, s_ref, mx_ref, den_ref, out_ref):
    t = pl.program_id(0)
    rows = t * _T + lax.broadcasted_iota(jnp.int32, (_T, 1), 0)
    in_seg = (rows >= starts_ref[...]) & (rows < ends_ref[...])
    mrow = jnp.sum(jnp.where(in_seg, mx_ref[...], 0.0), axis=1, keepdims=True)
    drow = jnp.sum(jnp.where(in_seg, den_ref[...], 0.0), axis=1, keepdims=True)
    out_ref[...] = jnp.exp(s_ref[...] - mrow) / drow


def _sc_stats_body(scores_hbm, offs_hbm, out_hbm, buf, offs_v, stat_v):
    w = lax.axis_index("s") * 2 + lax.axis_index("c")
    lane = lax.iota(jnp.int32, 16)

    wbase = pl.multiple_of(w * 16, 16)
    pltpu.sync_copy(offs_hbm.at[pl.ds(wbase, 16)], offs_v)
    ovec = offs_v[pl.ds(0, 16)]

    mvals = jnp.zeros((16,), jnp.float32)
    svals = jnp.zeros((16,), jnp.float32)
    for k in range(_SEG_PER_W):
        start = ovec[k]
        end = ovec[8 + k]
        astart = pl.multiple_of(start & ~7, 8)
        pltpu.sync_copy(scores_hbm.at[pl.ds(astart, _WIN)],
                        buf.at[pl.ds(0, _WIN)])

        mvec = jnp.full((16,), -3.0e38, jnp.float32)
        for c in range(_NCK):
            gl = astart + 16 * c + lane
            valid = (gl >= start) & (gl < end)
            mvec = jnp.maximum(mvec,
                               jnp.where(valid, buf[pl.ds(16 * c, 16)],
                                         -3.0e38))
        m = mvec[0]
        for i in range(1, 16):
            m = jnp.maximum(m, mvec[i])

        svec = jnp.zeros((16,), jnp.float32)
        for c in range(_NCK):
            gl = astart + 16 * c + lane
            valid = (gl >= start) & (gl < end)
            svec = svec + jnp.where(
                valid, jnp.exp(buf[pl.ds(16 * c, 16)] - m), 0.0)
        sm = svec[0]
        for i in range(1, 16):
            sm = sm + svec[i]

        mvals = jnp.where(lane == k, jnp.zeros((16,), jnp.float32) + m, mvals)
        svals = jnp.where(lane == (8 + k),
                          jnp.zeros((16,), jnp.float32) + sm, svals)

    stat_v[pl.ds(0, 16)] = jnp.where(lane < 8, mvals,
                                     jnp.zeros((16,), jnp.float32))
    stat_v[pl.ds(0, 16)] = stat_v[pl.ds(0, 16)] + jnp.where(
        lane >= 8, svals, jnp.zeros((16,), jnp.float32))
    pltpu.sync_copy(stat_v, out_hbm.at[pl.ds(wbase, 16)])


_sc_stats = functools.partial(
    pl.kernel,
    mesh=plsc.VectorSubcoreMesh(core_axis_name="c", subcore_axis_name="s"),
    out_type=jax.ShapeDtypeStruct((_NW * 16,), jnp.float32),
    scratch_types=[
        pltpu.VMEM((_WIN,), jnp.float32),
        pltpu.VMEM((16,), jnp.int32),
        pltpu.VMEM((16,), jnp.float32),
    ],
)(_sc_stats_body)


def kernel(prev_hidden_states, encoder_output, tree_sizes, W, b, v):
    w1t = W[:, :_HD].T.astype(jnp.bfloat16)
    w2t = W[:, _HD:].T.astype(jnp.bfloat16)
    csum = jnp.cumsum(tree_sizes.astype(jnp.int32))
    offs = jnp.concatenate([jnp.zeros((1,), jnp.int32), csum])
    starts = offs[:_B].reshape(1, _B)
    ends = offs[1:_B + 1].reshape(1, _B)
    seg_of_w = (jnp.arange(_NW)[:, None] +
                _NW * jnp.arange(_SEG_PER_W)[None, :])
    woffs = jnp.concatenate(
        [offs[seg_of_w], offs[seg_of_w + 1]], axis=1).reshape(_NW * 16)
    b2 = b.reshape(1, _HE)
    v16 = v.reshape(_HE, 1).astype(jnp.bfloat16)

    p_hi, p_lo = pl.pallas_call(
        _p_kernel,
        out_shape=[
            jax.ShapeDtypeStruct((_B, _HE), jnp.bfloat16),
            jax.ShapeDtypeStruct((_B, _HE), jnp.bfloat16),
        ],
    )(prev_hidden_states, w1t, b2)

    grid = _N // _T
    scores = pl.pallas_call(
        _scores_kernel,
        grid=(grid,),
        in_specs=[
            pl.BlockSpec((1, _B), lambda t: (0, 0)),
            pl.BlockSpec((1, _B), lambda t: (0, 0)),
            pl.BlockSpec((_T, _HE), lambda t: (t, 0)),
            pl.BlockSpec((_HE, _HE), lambda t: (0, 0)),
            pl.BlockSpec((_B, _HE), lambda t: (0, 0)),
            pl.BlockSpec((_B, _HE), lambda t: (0, 0)),
            pl.BlockSpec((_HE, 1), lambda t: (0, 0)),
        ],
        out_specs=pl.BlockSpec((_T, 1), lambda t: (t, 0)),
        out_shape=jax.ShapeDtypeStruct((_N, 1), jnp.float32),
    )(starts, ends, encoder_output, w2t, p_hi, p_lo, v16)

    spad = jnp.concatenate(
        [scores.reshape(_N), jnp.zeros((32,), jnp.float32)])
    stats = _sc_stats(spad, woffs).reshape(_NW, 16)
    mx_row = stats[:, :8].reshape(-1)[_SEG_PERM].reshape(1, _B)
    den_row = stats[:, 8:].reshape(-1)[_SEG_PERM].reshape(1, _B)

    att = pl.pallas_call(
        _norm_kernel,
        grid=(grid,),
        in_specs=[
            pl.BlockSpec((1, _B), lambda t: (0, 0)),
            pl.BlockSpec((1, _B), lambda t: (0, 0)),
            pl.BlockSpec((_T, 1), lambda t: (t, 0)),
            pl.BlockSpec((1, _B), lambda t: (0, 0)),
            pl.BlockSpec((1, _B), lambda t: (0, 0)),
        ],
        out_specs=pl.BlockSpec((_T, 1), lambda t: (t, 0)),
        out_shape=jax.ShapeDtypeStruct((_N, 1), jnp.float32),
    )(starts, ends, scores, mx_row, den_row)
    return att


# speedup vs baseline: 43.2834x; 1.0712x over previous
import functools

import numpy as np

import jax
import jax.numpy as jnp
from jax import lax
from jax.experimental import pallas as pl
from jax.experimental.pallas import tpu as pltpu
from jax.experimental.pallas import tpu_sc as plsc

_B = 256
_HE = 1024
_HD = 1024
_N = 32640
_T = 1088

_WIN = 272
_NCK = _WIN // 16
_NW = 32
_SEG_PER_W = _B // _NW
_SEG_PERM = np.argsort(
    (np.arange(_NW)[:, None] + _NW * np.arange(_SEG_PER_W)[None, :])
    .reshape(-1))


def _bdot(a, b):
    return jnp.dot(a, b, preferred_element_type=jnp.float32)


def _p_kernel(prev_ref, w1t_ref, b_ref, hi_ref, lo_ref):
    p = _bdot(prev_ref[...].astype(jnp.bfloat16), w1t_ref[...]) + b_ref[...]
    hi = p.astype(jnp.bfloat16)
    hi_ref[...] = hi
    lo_ref[...] = (p - hi.astype(jnp.float32)).astype(jnp.bfloat16)


def _scores_kernel(starts_ref, ends_ref, enc_ref, w2t_ref, phi_ref, plo_ref,
                   v_ref, out_ref):
    t = pl.program_id(0)
    rows = t * _T + lax.broadcasted_iota(jnp.int32, (_T, 1), 0)
    in_seg = (rows >= starts_ref[...]) & (rows < ends_ref[...])
    oh = in_seg.astype(jnp.bfloat16)
    pre = _bdot(enc_ref[...].astype(jnp.bfloat16), w2t_ref[...])
    pre = pre + (_bdot(oh, phi_ref[...]) + _bdot(oh, plo_ref[...]))
    energy = jnp.tanh(pre).astype(jnp.bfloat16)
    out_ref[...] = _bdot(energy, v_ref[...])


def _norm_kernel(starts_ref, ends_ref, s_ref, mx_ref, den_ref, out_ref):
    t = pl.program_id(0)
    rows = t * _T + lax.broadcasted_iota(jnp.int32, (_T, 1), 0)
    in_seg = (rows >= starts_ref[...]) & (rows < ends_ref[...])
    mrow = jnp.sum(jnp.where(in_seg, mx_ref[...], 0.0), axis=1, keepdims=True)
    drow = jnp.sum(jnp.where(in_seg, den_ref[...], 0.0), axis=1, keepdims=True)
    out_ref[...] = jnp.exp(s_ref[...] - mrow) / drow


def _sc_stats_body(scores_hbm, offs_hbm, out_hbm, buf, offs_v, stat_v):
    w = lax.axis_index("s") * 2 + lax.axis_index("c")
    lane = lax.iota(jnp.int32, 16)

    wbase = pl.multiple_of(w * 16, 16)
    pltpu.sync_copy(offs_hbm.at[pl.ds(wbase, 16)], offs_v)
    ovec = offs_v[pl.ds(0, 16)]

    mvals = jnp.zeros((16,), jnp.float32)
    svals = jnp.zeros((16,), jnp.float32)
    for k in range(_SEG_PER_W):
        start = ovec[k]
        end = ovec[8 + k]
        astart = pl.multiple_of(start & ~7, 8)
        pltpu.sync_copy(scores_hbm.at[pl.ds(astart, _WIN)],
                        buf.at[pl.ds(0, _WIN)])

        mvec = jnp.full((16,), -3.0e38, jnp.float32)
        for c in range(_NCK):
            gl = astart + 16 * c + lane
            valid = (gl >= start) & (gl < end)
            mvec = jnp.maximum(mvec,
                               jnp.where(valid, buf[pl.ds(16 * c, 16)],
                                         -3.0e38))
        m = mvec[0]
        for i in range(1, 16):
            m = jnp.maximum(m, mvec[i])

        svec = jnp.zeros((16,), jnp.float32)
        for c in range(_NCK):
            gl = astart + 16 * c + lane
            valid = (gl >= start) & (gl < end)
            svec = svec + jnp.where(
                valid, jnp.exp(buf[pl.ds(16 * c, 16)] - m), 0.0)
        sm = svec[0]
        for i in range(1, 16):
            sm = sm + svec[i]

        mvals = jnp.where(lane == k, jnp.zeros((16,), jnp.float32) + m, mvals)
        svals = jnp.where(lane == (8 + k),
                          jnp.zeros((16,), jnp.float32) + sm, svals)

    stat_v[pl.ds(0, 16)] = jnp.where(lane < 8, mvals,
                                     jnp.zeros((16,), jnp.float32))
    stat_v[pl.ds(0, 16)] = stat_v[pl.ds(0, 16)] + jnp.where(
        lane >= 8, svals, jnp.zeros((16,), jnp.float32))
    pltpu.sync_copy(stat_v, out_hbm.at[pl.ds(wbase, 16)])


_sc_stats = functools.partial(
    pl.kernel,
    mesh=plsc.VectorSubcoreMesh(core_axis_name="c", subcore_axis_name="s"),
    out_type=jax.ShapeDtypeStruct((_NW * 16,), jnp.float32),
    scratch_types=[
        pltpu.VMEM((_WIN,), jnp.float32),
        pltpu.VMEM((16,), jnp.int32),
        pltpu.VMEM((16,), jnp.float32),
    ],
)(_sc_stats_body)


def kernel(prev_hidden_states, encoder_output, tree_sizes, W, b, v):
    w1t = W[:, :_HD].T.astype(jnp.bfloat16)
    w2t = W[:, _HD:].T.astype(jnp.bfloat16)
    csum = jnp.cumsum(tree_sizes.astype(jnp.int32))
    offs = jnp.concatenate([jnp.zeros((1,), jnp.int32), csum])
    starts = offs[:_B].reshape(1, _B)
    ends = offs[1:_B + 1].reshape(1, _B)
    seg_of_w = (jnp.arange(_NW)[:, None] +
                _NW * jnp.arange(_SEG_PER_W)[None, :])
    woffs = jnp.concatenate(
        [offs[seg_of_w], offs[seg_of_w + 1]], axis=1).reshape(_NW * 16)
    b2 = b.reshape(1, _HE)
    v16 = v.reshape(_HE, 1).astype(jnp.bfloat16)

    p_hi, p_lo = pl.pallas_call(
        _p_kernel,
        out_shape=[
            jax.ShapeDtypeStruct((_B, _HE), jnp.bfloat16),
            jax.ShapeDtypeStruct((_B, _HE), jnp.bfloat16),
        ],
    )(prev_hidden_states, w1t, b2)

    grid = _N // _T
    scores = pl.pallas_call(
        _scores_kernel,
        grid=(grid,),
        in_specs=[
            pl.BlockSpec((1, _B), lambda t: (0, 0)),
            pl.BlockSpec((1, _B), lambda t: (0, 0)),
            pl.BlockSpec((_T, _HE), lambda t: (t, 0)),
            pl.BlockSpec((_HE, _HE), lambda t: (0, 0)),
            pl.BlockSpec((_B, _HE), lambda t: (0, 0)),
            pl.BlockSpec((_B, _HE), lambda t: (0, 0)),
            pl.BlockSpec((_HE, 1), lambda t: (0, 0)),
        ],
        out_specs=pl.BlockSpec((_T, 1), lambda t: (t, 0)),
        out_shape=jax.ShapeDtypeStruct((_N, 1), jnp.float32),
    )(starts, ends, encoder_output, w2t, p_hi, p_lo, v16)

    spad = jnp.concatenate(
        [scores.reshape(_N), jnp.zeros((32,), jnp.float32)])
    stats = _sc_stats(spad, woffs).reshape(_NW, 16)
    mx_row = stats[:, :8].reshape(-1)[_SEG_PERM].reshape(1, _B)
    den_row = stats[:, 8:].reshape(-1)[_SEG_PERM].reshape(1, _B)

    att = pl.pallas_call(
        _norm_kernel,
        grid=(grid,),
        in_specs=[
            pl.BlockSpec((1, _B), lambda t: (0, 0)),
            pl.BlockSpec((1, _B), lambda t: (0, 0)),
            pl.BlockSpec((_T, 1), lambda t: (t, 0)),
            pl.BlockSpec((1, _B), lambda t: (0, 0)),
            pl.BlockSpec((1, _B), lambda t: (0, 0)),
        ],
        out_specs=pl.BlockSpec((_T, 1), lambda t: (t, 0)),
        out_shape=jax.ShapeDtypeStruct((_N, 1), jnp.float32),
    )(starts, ends, scores, mx_row, den_row)
    return att


# speedup vs baseline: 45.4236x; 1.0494x over previous
import functools

import numpy as np

import jax
import jax.numpy as jnp
from jax import lax
from jax.experimental import pallas as pl
from jax.experimental.pallas import tpu as pltpu
from jax.experimental.pallas import tpu_sc as plsc

_B = 256
_HE = 1024
_HD = 1024
_N = 32640
_T = 2176

_WIN = 272
_NCK = _WIN // 16
_NW = 32
_SEG_PER_W = _B // _NW
_SEG_PERM = np.argsort(
    (np.arange(_NW)[:, None] + _NW * np.arange(_SEG_PER_W)[None, :])
    .reshape(-1))


def _bdot(a, b):
    return jnp.dot(a, b, preferred_element_type=jnp.float32)


def _p_kernel(prev_ref, w1t_ref, b_ref, hi_ref, lo_ref):
    p = _bdot(prev_ref[...].astype(jnp.bfloat16), w1t_ref[...]) + b_ref[...]
    hi = p.astype(jnp.bfloat16)
    hi_ref[...] = hi
    lo_ref[...] = (p - hi.astype(jnp.float32)).astype(jnp.bfloat16)


def _scores_kernel(starts_ref, ends_ref, enc_ref, w2t_ref, phi_ref, plo_ref,
                   v_ref, out_ref):
    t = pl.program_id(0)
    rows = t * _T + lax.broadcasted_iota(jnp.int32, (_T, 1), 0)
    in_seg = (rows >= starts_ref[...]) & (rows < ends_ref[...])
    oh = in_seg.astype(jnp.bfloat16)
    pre = _bdot(enc_ref[...].astype(jnp.bfloat16), w2t_ref[...])
    pre = pre + (_bdot(oh, phi_ref[...]) + _bdot(oh, plo_ref[...]))
    energy = jnp.tanh(pre).astype(jnp.bfloat16)
    out_ref[...] = _bdot(energy, v_ref[...])


def _norm_kernel(starts_ref, ends_ref, s_ref, mx_ref, den_ref, out_ref):
    t = pl.program_id(0)
    rows = t * _T + lax.broadcasted_iota(jnp.int32, (_T, 1), 0)
    in_seg = (rows >= starts_ref[...]) & (rows < ends_ref[...])
    mrow = jnp.sum(jnp.where(in_seg, mx_ref[...], 0.0), axis=1, keepdims=True)
    drow = jnp.sum(jnp.where(in_seg, den_ref[...], 0.0), axis=1, keepdims=True)
    out_ref[...] = jnp.exp(s_ref[...] - mrow) / drow


def _sc_stats_body(scores_hbm, offs_hbm, out_hbm, buf, offs_v, stat_v):
    w = lax.axis_index("s") * 2 + lax.axis_index("c")
    lane = lax.iota(jnp.int32, 16)

    wbase = pl.multiple_of(w * 16, 16)
    pltpu.sync_copy(offs_hbm.at[pl.ds(wbase, 16)], offs_v)
    ovec = offs_v[pl.ds(0, 16)]

    mvals = jnp.zeros((16,), jnp.float32)
    svals = jnp.zeros((16,), jnp.float32)
    for k in range(_SEG_PER_W):
        start = ovec[k]
        end = ovec[8 + k]
        astart = pl.multiple_of(start & ~7, 8)
        pltpu.sync_copy(scores_hbm.at[pl.ds(astart, _WIN)],
                        buf.at[pl.ds(0, _WIN)])

        mvec = jnp.full((16,), -3.0e38, jnp.float32)
        for c in range(_NCK):
            gl = astart + 16 * c + lane
            valid = (gl >= start) & (gl < end)
            mvec = jnp.maximum(mvec,
                               jnp.where(valid, buf[pl.ds(16 * c, 16)],
                                         -3.0e38))
        m = mvec[0]
        for i in range(1, 16):
            m = jnp.maximum(m, mvec[i])

        svec = jnp.zeros((16,), jnp.float32)
        for c in range(_NCK):
            gl = astart + 16 * c + lane
            valid = (gl >= start) & (gl < end)
            svec = svec + jnp.where(
                valid, jnp.exp(buf[pl.ds(16 * c, 16)] - m), 0.0)
        sm = svec[0]
        for i in range(1, 16):
            sm = sm + svec[i]

        mvals = jnp.where(lane == k, jnp.zeros((16,), jnp.float32) + m, mvals)
        svals = jnp.where(lane == (8 + k),
                          jnp.zeros((16,), jnp.float32) + sm, svals)

    stat_v[pl.ds(0, 16)] = jnp.where(lane < 8, mvals,
                                     jnp.zeros((16,), jnp.float32))
    stat_v[pl.ds(0, 16)] = stat_v[pl.ds(0, 16)] + jnp.where(
        lane >= 8, svals, jnp.zeros((16,), jnp.float32))
    pltpu.sync_copy(stat_v, out_hbm.at[pl.ds(wbase, 16)])


_sc_stats = functools.partial(
    pl.kernel,
    mesh=plsc.VectorSubcoreMesh(core_axis_name="c", subcore_axis_name="s"),
    out_type=jax.ShapeDtypeStruct((_NW * 16,), jnp.float32),
    scratch_types=[
        pltpu.VMEM((_WIN,), jnp.float32),
        pltpu.VMEM((16,), jnp.int32),
        pltpu.VMEM((16,), jnp.float32),
    ],
)(_sc_stats_body)


def kernel(prev_hidden_states, encoder_output, tree_sizes, W, b, v):
    w1t = W[:, :_HD].T.astype(jnp.bfloat16)
    w2t = W[:, _HD:].T.astype(jnp.bfloat16)
    csum = jnp.cumsum(tree_sizes.astype(jnp.int32))
    offs = jnp.concatenate([jnp.zeros((1,), jnp.int32), csum])
    starts = offs[:_B].reshape(1, _B)
    ends = offs[1:_B + 1].reshape(1, _B)
    seg_of_w = (jnp.arange(_NW)[:, None] +
                _NW * jnp.arange(_SEG_PER_W)[None, :])
    woffs = jnp.concatenate(
        [offs[seg_of_w], offs[seg_of_w + 1]], axis=1).reshape(_NW * 16)
    b2 = b.reshape(1, _HE)
    v16 = v.reshape(_HE, 1).astype(jnp.bfloat16)

    p_hi, p_lo = pl.pallas_call(
        _p_kernel,
        out_shape=[
            jax.ShapeDtypeStruct((_B, _HE), jnp.bfloat16),
            jax.ShapeDtypeStruct((_B, _HE), jnp.bfloat16),
        ],
    )(prev_hidden_states, w1t, b2)

    grid = _N // _T
    scores = pl.pallas_call(
        _scores_kernel,
        grid=(grid,),
        in_specs=[
            pl.BlockSpec((1, _B), lambda t: (0, 0)),
            pl.BlockSpec((1, _B), lambda t: (0, 0)),
            pl.BlockSpec((_T, _HE), lambda t: (t, 0)),
            pl.BlockSpec((_HE, _HE), lambda t: (0, 0)),
            pl.BlockSpec((_B, _HE), lambda t: (0, 0)),
            pl.BlockSpec((_B, _HE), lambda t: (0, 0)),
            pl.BlockSpec((_HE, 1), lambda t: (0, 0)),
        ],
        out_specs=pl.BlockSpec((_T, 1), lambda t: (t, 0)),
        out_shape=jax.ShapeDtypeStruct((_N, 1), jnp.float32),
    )(starts, ends, encoder_output, w2t, p_hi, p_lo, v16)

    spad = jnp.concatenate(
        [scores.reshape(_N), jnp.zeros((32,), jnp.float32)])
    stats = _sc_stats(spad, woffs).reshape(_NW, 16)
    mx_row = stats[:, :8].reshape(-1)[_SEG_PERM].reshape(1, _B)
    den_row = stats[:, 8:].reshape(-1)[_SEG_PERM].reshape(1, _B)

    att = pl.pallas_call(
        _norm_kernel,
        grid=(grid,),
        in_specs=[
            pl.BlockSpec((1, _B), lambda t: (0, 0)),
            pl.BlockSpec((1, _B), lambda t: (0, 0)),
            pl.BlockSpec((_T, 1), lambda t: (t, 0)),
            pl.BlockSpec((1, _B), lambda t: (0, 0)),
            pl.BlockSpec((1, _B), lambda t: (0, 0)),
        ],
        out_specs=pl.BlockSpec((_T, 1), lambda t: (t, 0)),
        out_shape=jax.ShapeDtypeStruct((_N, 1), jnp.float32),
    )(starts, ends, scores, mx_row, den_row)
    return att


# speedup vs baseline: 46.0519x; 1.0138x over previous
import functools

import numpy as np

import jax
import jax.numpy as jnp
from jax import lax
from jax.experimental import pallas as pl
from jax.experimental.pallas import tpu as pltpu
from jax.experimental.pallas import tpu_sc as plsc

_B = 256
_HE = 1024
_HD = 1024
_N = 32640
_T = 3264

_WIN = 272
_NCK = _WIN // 16
_NW = 32
_SEG_PER_W = _B // _NW
_SEG_PERM = np.argsort(
    (np.arange(_NW)[:, None] + _NW * np.arange(_SEG_PER_W)[None, :])
    .reshape(-1))


def _bdot(a, b):
    return jnp.dot(a, b, preferred_element_type=jnp.float32)


def _p_kernel(prev_ref, w1t_ref, b_ref, hi_ref, lo_ref):
    p = _bdot(prev_ref[...].astype(jnp.bfloat16), w1t_ref[...]) + b_ref[...]
    hi = p.astype(jnp.bfloat16)
    hi_ref[...] = hi
    lo_ref[...] = (p - hi.astype(jnp.float32)).astype(jnp.bfloat16)


def _scores_kernel(starts_ref, ends_ref, enc_ref, w2t_ref, phi_ref, plo_ref,
                   v_ref, out_ref):
    t = pl.program_id(0)
    rows = t * _T + lax.broadcasted_iota(jnp.int32, (_T, 1), 0)
    in_seg = (rows >= starts_ref[...]) & (rows < ends_ref[...])
    oh = in_seg.astype(jnp.bfloat16)
    pre = _bdot(enc_ref[...].astype(jnp.bfloat16), w2t_ref[...])
    pre = pre + (_bdot(oh, phi_ref[...]) + _bdot(oh, plo_ref[...]))
    energy = jnp.tanh(pre).astype(jnp.bfloat16)
    out_ref[...] = _bdot(energy, v_ref[...])


def _norm_kernel(starts_ref, ends_ref, s_ref, mx_ref, den_ref, out_ref):
    t = pl.program_id(0)
    rows = t * _T + lax.broadcasted_iota(jnp.int32, (_T, 1), 0)
    in_seg = (rows >= starts_ref[...]) & (rows < ends_ref[...])
    mrow = jnp.sum(jnp.where(in_seg, mx_ref[...], 0.0), axis=1, keepdims=True)
    drow = jnp.sum(jnp.where(in_seg, den_ref[...], 0.0), axis=1, keepdims=True)
    out_ref[...] = jnp.exp(s_ref[...] - mrow) / drow


def _sc_stats_body(scores_hbm, offs_hbm, out_hbm, buf, offs_v, stat_v):
    w = lax.axis_index("s") * 2 + lax.axis_index("c")
    lane = lax.iota(jnp.int32, 16)

    wbase = pl.multiple_of(w * 16, 16)
    pltpu.sync_copy(offs_hbm.at[pl.ds(wbase, 16)], offs_v)
    ovec = offs_v[pl.ds(0, 16)]

    mvals = jnp.zeros((16,), jnp.float32)
    svals = jnp.zeros((16,), jnp.float32)
    for k in range(_SEG_PER_W):
        start = ovec[k]
        end = ovec[8 + k]
        astart = pl.multiple_of(start & ~7, 8)
        pltpu.sync_copy(scores_hbm.at[pl.ds(astart, _WIN)],
                        buf.at[pl.ds(0, _WIN)])

        mvec = jnp.full((16,), -3.0e38, jnp.float32)
        for c in range(_NCK):
            gl = astart + 16 * c + lane
            valid = (gl >= start) & (gl < end)
            mvec = jnp.maximum(mvec,
                               jnp.where(valid, buf[pl.ds(16 * c, 16)],
                                         -3.0e38))
        m = mvec[0]
        for i in range(1, 16):
            m = jnp.maximum(m, mvec[i])

        svec = jnp.zeros((16,), jnp.float32)
        for c in range(_NCK):
            gl = astart + 16 * c + lane
            valid = (gl >= start) & (gl < end)
            svec = svec + jnp.where(
                valid, jnp.exp(buf[pl.ds(16 * c, 16)] - m), 0.0)
        sm = svec[0]
        for i in range(1, 16):
            sm = sm + svec[i]

        mvals = jnp.where(lane == k, jnp.zeros((16,), jnp.float32) + m, mvals)
        svals = jnp.where(lane == (8 + k),
                          jnp.zeros((16,), jnp.float32) + sm, svals)

    stat_v[pl.ds(0, 16)] = jnp.where(lane < 8, mvals,
                                     jnp.zeros((16,), jnp.float32))
    stat_v[pl.ds(0, 16)] = stat_v[pl.ds(0, 16)] + jnp.where(
        lane >= 8, svals, jnp.zeros((16,), jnp.float32))
    pltpu.sync_copy(stat_v, out_hbm.at[pl.ds(wbase, 16)])


_sc_stats = functools.partial(
    pl.kernel,
    mesh=plsc.VectorSubcoreMesh(core_axis_name="c", subcore_axis_name="s"),
    out_type=jax.ShapeDtypeStruct((_NW * 16,), jnp.float32),
    scratch_types=[
        pltpu.VMEM((_WIN,), jnp.float32),
        pltpu.VMEM((16,), jnp.int32),
        pltpu.VMEM((16,), jnp.float32),
    ],
)(_sc_stats_body)


def kernel(prev_hidden_states, encoder_output, tree_sizes, W, b, v):
    w1t = W[:, :_HD].T.astype(jnp.bfloat16)
    w2t = W[:, _HD:].T.astype(jnp.bfloat16)
    csum = jnp.cumsum(tree_sizes.astype(jnp.int32))
    offs = jnp.concatenate([jnp.zeros((1,), jnp.int32), csum])
    starts = offs[:_B].reshape(1, _B)
    ends = offs[1:_B + 1].reshape(1, _B)
    seg_of_w = (jnp.arange(_NW)[:, None] +
                _NW * jnp.arange(_SEG_PER_W)[None, :])
    woffs = jnp.concatenate(
        [offs[seg_of_w], offs[seg_of_w + 1]], axis=1).reshape(_NW * 16)
    b2 = b.reshape(1, _HE)
    v16 = v.reshape(_HE, 1).astype(jnp.bfloat16)

    p_hi, p_lo = pl.pallas_call(
        _p_kernel,
        out_shape=[
            jax.ShapeDtypeStruct((_B, _HE), jnp.bfloat16),
            jax.ShapeDtypeStruct((_B, _HE), jnp.bfloat16),
        ],
    )(prev_hidden_states, w1t, b2)

    grid = _N // _T
    scores = pl.pallas_call(
        _scores_kernel,
        grid=(grid,),
        in_specs=[
            pl.BlockSpec((1, _B), lambda t: (0, 0)),
            pl.BlockSpec((1, _B), lambda t: (0, 0)),
            pl.BlockSpec((_T, _HE), lambda t: (t, 0)),
            pl.BlockSpec((_HE, _HE), lambda t: (0, 0)),
            pl.BlockSpec((_B, _HE), lambda t: (0, 0)),
            pl.BlockSpec((_B, _HE), lambda t: (0, 0)),
            pl.BlockSpec((_HE, 1), lambda t: (0, 0)),
        ],
        out_specs=pl.BlockSpec((_T, 1), lambda t: (t, 0)),
        out_shape=jax.ShapeDtypeStruct((_N, 1), jnp.float32),
    )(starts, ends, encoder_output, w2t, p_hi, p_lo, v16)

    spad = jnp.concatenate(
        [scores.reshape(_N), jnp.zeros((32,), jnp.float32)])
    stats = _sc_stats(spad, woffs).reshape(_NW, 16)
    mx_row = stats[:, :8].reshape(-1)[_SEG_PERM].reshape(1, _B)
    den_row = stats[:, 8:].reshape(-1)[_SEG_PERM].reshape(1, _B)

    att = pl.pallas_call(
        _norm_kernel,
        grid=(grid,),
        in_specs=[
            pl.BlockSpec((1, _B), lambda t: (0, 0)),
            pl.BlockSpec((1, _B), lambda t: (0, 0)),
            pl.BlockSpec((_T, 1), lambda t: (t, 0)),
            pl.BlockSpec((1, _B), lambda t: (0, 0)),
            pl.BlockSpec((1, _B), lambda t: (0, 0)),
        ],
        out_specs=pl.BlockSpec((_T, 1), lambda t: (t, 0)),
        out_shape=jax.ShapeDtypeStruct((_N, 1), jnp.float32),
    )(starts, ends, scores, mx_row, den_row)
    return att
